# baseline (speedup 1.0000x reference)
import functools

import jax
import jax.numpy as jnp
from jax import lax
from jax.experimental import pallas as pl
from jax.experimental.pallas import tpu as pltpu
from jax.experimental.pallas import tpu_sc as plsc

B = 16384
D = 32
L = 16
NC = 2
NS = 16
NW = NC * NS
BPW = B // NW
CHUNK = 128
NCHUNK = BPW // CHUNK

_mesh = plsc.VectorSubcoreMesh(core_axis_name="c", subcore_axis_name="s")


@functools.partial(
    pl.kernel,
    mesh=_mesh,
    compiler_params=pltpu.CompilerParams(
        needs_layout_passes=False, use_tc_tiling_on_sc=False),
    out_type=jax.ShapeDtypeStruct((B,), jnp.float32),
    scratch_types=[
        pltpu.VMEM((NCHUNK, CHUNK), jnp.int32),
        pltpu.VMEM((NCHUNK, CHUNK), jnp.int32),
        pltpu.VMEM((BPW, D), jnp.float32),
        pltpu.VMEM((BPW, D), jnp.float32),
        pltpu.VMEM((BPW,), jnp.float32),
        pltpu.VMEM((L * (L + 1),), jnp.float32),
        pltpu.SemaphoreType.DMA,
        pltpu.SemaphoreType.DMA,
    ],
)
def _sc_forward(users_hbm, items_hbm, ut_hbm, it_hbm, out_hbm,
                uidx_v, iidx_v, urows_v, irows_v, out_v, t_v, sem_u, sem_i):
    wid = lax.axis_index("s") * NC + lax.axis_index("c")
    base = wid * BPW

    for c in range(NCHUNK):
        pltpu.sync_copy(users_hbm.at[pl.ds(base + c * CHUNK, CHUNK)],
                        uidx_v.at[c])
        pltpu.sync_copy(items_hbm.at[pl.ds(base + c * CHUNK, CHUNK)],
                        iidx_v.at[c])

    copies = []
    for c in range(NCHUNK):
        copies.append(pltpu.async_copy(
            ut_hbm.at[uidx_v.at[c]],
            urows_v.at[pl.ds(c * CHUNK, CHUNK)], sem_u))
        copies.append(pltpu.async_copy(
            it_hbm.at[iidx_v.at[c]],
            irows_v.at[pl.ds(c * CHUNK, CHUNK)], sem_i))
    for cp in copies:
        cp.wait()

    lanes17 = lax.iota(jnp.int32, L) * (L + 1)

    def group_body(g, _):
        row0 = g * L
        for j in range(L):
            b = row0 + j
            lo = urows_v[b, pl.ds(0, L)] * irows_v[b, pl.ds(0, L)]
            hi = urows_v[b, pl.ds(L, L)] * irows_v[b, pl.ds(L, L)]
            plsc.store_scatter(t_v, [lanes17 + j], lo + hi)
        acc = t_v[pl.ds(0, L)]
        for l in range(1, L):
            acc = acc + t_v[pl.ds(l * (L + 1), L)]
        out_v[pl.ds(row0, L)] = acc
        return 0

    lax.fori_loop(0, BPW // L, group_body, 0)

    pltpu.sync_copy(out_v, out_hbm.at[pl.ds(base, BPW)])


def kernel(users, items, user_table, item_table):
    return _sc_forward(users, items, user_table, item_table)


# speedup vs baseline: 1.4547x; 1.4547x over previous
import functools

import jax
import jax.numpy as jnp
from jax import lax
from jax.experimental import pallas as pl
from jax.experimental.pallas import tpu as pltpu
from jax.experimental.pallas import tpu_sc as plsc

B = 16384
D = 32
L = 16
NC = 2
NS = 16
NW = NC * NS
BPW = B // NW
NG = BPW // L

_mesh = plsc.VectorSubcoreMesh(core_axis_name="c", subcore_axis_name="s")


@functools.partial(
    pl.kernel,
    mesh=_mesh,
    compiler_params=pltpu.CompilerParams(needs_layout_passes=False),
    out_type=jax.ShapeDtypeStruct((B,), jnp.float32),
    scratch_types=[
        pltpu.VMEM((BPW,), jnp.int32),
        pltpu.VMEM((BPW,), jnp.int32),
        pltpu.VMEM((L, D), jnp.float32),
        pltpu.VMEM((L, D), jnp.float32),
        pltpu.VMEM((BPW,), jnp.float32),
        pltpu.VMEM((L * (L + 1),), jnp.float32),
        pltpu.SemaphoreType.DMA,
    ],
)
def _sc_forward(users_hbm, items_hbm, ut_hbm, it_hbm, out_hbm,
                uidx_v, iidx_v, urows_v, irows_v, out_v, t_v, sem):
    wid = lax.axis_index("s") * NC + lax.axis_index("c")
    base = wid * BPW

    pltpu.sync_copy(users_hbm.at[pl.ds(base, BPW)], uidx_v)
    pltpu.sync_copy(items_hbm.at[pl.ds(base, BPW)], iidx_v)

    lanes17 = lax.iota(jnp.int32, L) * (L + 1)

    def group_body(g, _):
        row0 = g * L
        uv = uidx_v[pl.ds(row0, L)]
        iv = iidx_v[pl.ds(row0, L)]
        copies = []
        for j in range(L):
            copies.append(pltpu.async_copy(
                ut_hbm.at[uv[j]], urows_v.at[j], sem))
            copies.append(pltpu.async_copy(
                it_hbm.at[iv[j]], irows_v.at[j], sem))
        for cp in copies:
            cp.wait()
        for j in range(L):
            lo = urows_v[j, pl.ds(0, L)] * irows_v[j, pl.ds(0, L)]
            hi = urows_v[j, pl.ds(L, L)] * irows_v[j, pl.ds(L, L)]
            plsc.store_scatter(t_v, [lanes17 + j], lo + hi)
        acc = t_v[pl.ds(0, L)]
        for l in range(1, L):
            acc = acc + t_v[pl.ds(l * (L + 1), L)]
        out_v[pl.ds(row0, L)] = acc
        return 0

    lax.fori_loop(0, NG, group_body, 0)

    pltpu.sync_copy(out_v, out_hbm.at[pl.ds(base, BPW)])


def kernel(users, items, user_table, item_table):
    return _sc_forward(users, items, user_table, item_table)


# speedup vs baseline: 2.3188x; 1.5939x over previous
import functools

import jax
import jax.numpy as jnp
from jax import lax
from jax.experimental import pallas as pl
from jax.experimental.pallas import tpu as pltpu
from jax.experimental.pallas import tpu_sc as plsc

B = 16384
D = 32
L = 16
NC = 2
NS = 16
NW = NC * NS
BPW = B // NW
NG = BPW // L
NT = 125000

_mesh = plsc.VectorSubcoreMesh(core_axis_name="c", subcore_axis_name="s")


@functools.partial(
    pl.kernel,
    mesh=_mesh,
    compiler_params=pltpu.CompilerParams(needs_layout_passes=False),
    out_type=jax.ShapeDtypeStruct((B,), jnp.float32),
    scratch_types=[
        pltpu.VMEM((BPW,), jnp.int32),
        pltpu.VMEM((BPW,), jnp.int32),
        pltpu.VMEM((2, L, 8, D), jnp.float32),
        pltpu.VMEM((2, L, 8, D), jnp.float32),
        pltpu.VMEM((BPW,), jnp.float32),
        pltpu.VMEM((L * (L + 1),), jnp.float32),
        pltpu.SemaphoreType.DMA,
        pltpu.SemaphoreType.DMA,
    ],
)
def _sc_forward(users_hbm, items_hbm, ut_hbm, it_hbm, out_hbm,
                uidx_v, iidx_v, ubuf_v, ibuf_v, out_v, t_v, sem_u, sem_i):
    wid = lax.axis_index("s") * NC + lax.axis_index("c")
    base = wid * BPW

    pltpu.sync_copy(users_hbm.at[pl.ds(base, BPW)], uidx_v)
    pltpu.sync_copy(items_hbm.at[pl.ds(base, BPW)], iidx_v)

    lanes17 = lax.iota(jnp.int32, L) * (L + 1)

    def fire(g, slot):
        ut = jax.lax.shift_right_logical(uidx_v[pl.ds(g * L, L)], 3)
        it = jax.lax.shift_right_logical(iidx_v[pl.ds(g * L, L)], 3)
        for j in range(L):
            pltpu.async_copy(ut_hbm.at[ut[j]], ubuf_v.at[slot, j], sem_u)
            pltpu.async_copy(it_hbm.at[it[j]], ibuf_v.at[slot, j], sem_i)

    def drain(slot):
        for j in range(L):
            pltpu.make_async_copy(
                ut_hbm.at[0], ubuf_v.at[slot, j], sem_u).wait()
            pltpu.make_async_copy(
                it_hbm.at[0], ibuf_v.at[slot, j], sem_i).wait()

    def compute(g, slot):
        row0 = g * L
        uv = uidx_v[pl.ds(row0, L)] & 7
        iv = iidx_v[pl.ds(row0, L)] & 7
        for j in range(L):
            su = uv[j]
            si = iv[j]
            lo = (ubuf_v[slot, j, su, pl.ds(0, L)]
                  * ibuf_v[slot, j, si, pl.ds(0, L)])
            hi = (ubuf_v[slot, j, su, pl.ds(L, L)]
                  * ibuf_v[slot, j, si, pl.ds(L, L)])
            plsc.store_scatter(t_v, [lanes17 + j], lo + hi)
        acc = t_v[pl.ds(0, L)]
        for l in range(1, L):
            acc = acc + t_v[pl.ds(l * (L + 1), L)]
        out_v[pl.ds(row0, L)] = acc

    fire(0, 0)

    def pair_body(h, carry):
        g0 = 2 * h
        fire(g0 + 1, 1)
        drain(0)
        compute(g0, 0)
        fire(lax.rem(g0 + 2, NG), 0)
        drain(1)
        compute(g0 + 1, 1)
        return carry

    lax.fori_loop(0, NG // 2, pair_body, 0)
    drain(0)

    pltpu.sync_copy(out_v, out_hbm.at[pl.ds(base, BPW)])


def kernel(users, items, user_table, item_table):
    ut3 = user_table.reshape(NT, 8, D)
    it3 = item_table.reshape(NT, 8, D)
    return _sc_forward(users, items, ut3, it3)


# speedup vs baseline: 2.4549x; 1.0587x over previous
import functools

import jax
import jax.numpy as jnp
from jax import lax
from jax.experimental import pallas as pl
from jax.experimental.pallas import tpu as pltpu
from jax.experimental.pallas import tpu_sc as plsc

B = 16384
D = 32
L = 16
NC = 2
NS = 16
NW = NC * NS
BPW = B // NW
NG = BPW // L
NT = 125000

_mesh = plsc.VectorSubcoreMesh(core_axis_name="c", subcore_axis_name="s")


@functools.partial(
    pl.kernel,
    mesh=_mesh,
    compiler_params=pltpu.CompilerParams(needs_layout_passes=False),
    out_type=jax.ShapeDtypeStruct((B,), jnp.float32),
    scratch_types=[
        pltpu.VMEM((BPW,), jnp.int32),
        pltpu.VMEM((BPW,), jnp.int32),
        pltpu.VMEM((2, L, 4, D), jnp.float32),
        pltpu.VMEM((2, L, 4, D), jnp.float32),
        pltpu.VMEM((BPW,), jnp.float32),
        pltpu.VMEM((L * (L + 1),), jnp.float32),
        pltpu.SemaphoreType.DMA,
        pltpu.SemaphoreType.DMA,
    ],
)
def _sc_forward(users_hbm, items_hbm, ut_hbm, it_hbm, out_hbm,
                uidx_v, iidx_v, ubuf_v, ibuf_v, out_v, t_v, sem_u, sem_i):
    wid = lax.axis_index("s") * NC + lax.axis_index("c")
    base = wid * BPW

    pltpu.sync_copy(users_hbm.at[pl.ds(base, BPW)], uidx_v)
    pltpu.sync_copy(items_hbm.at[pl.ds(base, BPW)], iidx_v)

    lanes17 = lax.iota(jnp.int32, L) * (L + 1)

    def fire(g, slot):
        uvg = uidx_v[pl.ds(g * L, L)]
        ivg = iidx_v[pl.ds(g * L, L)]
        ut = jax.lax.shift_right_logical(uvg, 3)
        it = jax.lax.shift_right_logical(ivg, 3)
        uq = jax.lax.shift_right_logical(uvg, 2) & 1
        iq = jax.lax.shift_right_logical(ivg, 2) & 1
        for j in range(L):
            pltpu.async_copy(
                ut_hbm.at[ut[j], pl.ds(uq[j] * 4, 4), :],
                ubuf_v.at[slot, j], sem_u)
            pltpu.async_copy(
                it_hbm.at[it[j], pl.ds(iq[j] * 4, 4), :],
                ibuf_v.at[slot, j], sem_i)

    def drain(slot):
        for j in range(L):
            pltpu.make_async_copy(
                ut_hbm.at[0, pl.ds(0, 4), :], ubuf_v.at[slot, j],
                sem_u).wait()
            pltpu.make_async_copy(
                it_hbm.at[0, pl.ds(0, 4), :], ibuf_v.at[slot, j],
                sem_i).wait()

    def compute(g, slot):
        row0 = g * L
        uv = uidx_v[pl.ds(row0, L)] & 3
        iv = iidx_v[pl.ds(row0, L)] & 3
        for j in range(L):
            su = uv[j]
            si = iv[j]
            lo = (ubuf_v[slot, j, su, pl.ds(0, L)]
                  * ibuf_v[slot, j, si, pl.ds(0, L)])
            hi = (ubuf_v[slot, j, su, pl.ds(L, L)]
                  * ibuf_v[slot, j, si, pl.ds(L, L)])
            plsc.store_scatter(t_v, [lanes17 + j], lo + hi)
        acc = t_v[pl.ds(0, L)]
        for l in range(1, L):
            acc = acc + t_v[pl.ds(l * (L + 1), L)]
        out_v[pl.ds(row0, L)] = acc

    fire(0, 0)

    def pair_body(h, carry):
        g0 = 2 * h
        fire(g0 + 1, 1)
        drain(0)
        compute(g0, 0)
        fire(lax.rem(g0 + 2, NG), 0)
        drain(1)
        compute(g0 + 1, 1)
        return carry

    lax.fori_loop(0, NG // 2, pair_body, 0)
    drain(0)

    pltpu.sync_copy(out_v, out_hbm.at[pl.ds(base, BPW)])


def kernel(users, items, user_table, item_table):
    ut3 = user_table.reshape(NT, 8, D)
    it3 = item_table.reshape(NT, 8, D)
    return _sc_forward(users, items, ut3, it3)


# speedup vs baseline: 2.5590x; 1.0424x over previous
import functools

import jax
import jax.numpy as jnp
from jax import lax
from jax.experimental import pallas as pl
from jax.experimental.pallas import tpu as pltpu
from jax.experimental.pallas import tpu_sc as plsc

B = 16384
D = 32
L = 16
NC = 2
NS = 16
NW = NC * NS
BPW = B // NW
NG = BPW // L
NT = 125000

_mesh = plsc.VectorSubcoreMesh(core_axis_name="c", subcore_axis_name="s")


@functools.partial(
    pl.kernel,
    mesh=_mesh,
    compiler_params=pltpu.CompilerParams(needs_layout_passes=False),
    out_type=jax.ShapeDtypeStruct((B,), jnp.float32),
    scratch_types=[
        pltpu.VMEM((BPW,), jnp.int32),
        pltpu.VMEM((BPW,), jnp.int32),
        pltpu.VMEM((2, L, D), jnp.float32),
        pltpu.VMEM((2, L, D), jnp.float32),
        pltpu.VMEM((BPW,), jnp.float32),
        pltpu.VMEM((L * (L + 1),), jnp.float32),
        pltpu.SemaphoreType.DMA,
        pltpu.SemaphoreType.DMA,
    ],
)
def _sc_forward(users_hbm, items_hbm, ut_hbm, it_hbm, out_hbm,
                uidx_v, iidx_v, ubuf_v, ibuf_v, out_v, t_v, sem_u, sem_i):
    wid = lax.axis_index("s") * NC + lax.axis_index("c")
    base = wid * BPW

    pltpu.sync_copy(users_hbm.at[pl.ds(base, BPW)], uidx_v)
    pltpu.sync_copy(items_hbm.at[pl.ds(base, BPW)], iidx_v)

    lanes17 = lax.iota(jnp.int32, L) * (L + 1)

    def fire(g, slot):
        uvg = uidx_v[pl.ds(g * L, L)]
        ivg = iidx_v[pl.ds(g * L, L)]
        ut = jax.lax.shift_right_logical(uvg, 3)
        it = jax.lax.shift_right_logical(ivg, 3)
        us = uvg & 7
        ws = ivg & 7
        for j in range(L):
            pltpu.async_copy(
                ut_hbm.at[ut[j], us[j], :], ubuf_v.at[slot, j], sem_u)
            pltpu.async_copy(
                it_hbm.at[it[j], ws[j], :], ibuf_v.at[slot, j], sem_i)

    def drain(slot):
        for j in range(L):
            pltpu.make_async_copy(
                ut_hbm.at[0, 0, :], ubuf_v.at[slot, j], sem_u).wait()
            pltpu.make_async_copy(
                it_hbm.at[0, 0, :], ibuf_v.at[slot, j], sem_i).wait()

    def compute(g, slot):
        row0 = g * L
        for j in range(L):
            lo = (ubuf_v[slot, j, pl.ds(0, L)]
                  * ibuf_v[slot, j, pl.ds(0, L)])
            hi = (ubuf_v[slot, j, pl.ds(L, L)]
                  * ibuf_v[slot, j, pl.ds(L, L)])
            plsc.store_scatter(t_v, [lanes17 + j], lo + hi)
        acc = t_v[pl.ds(0, L)]
        for l in range(1, L):
            acc = acc + t_v[pl.ds(l * (L + 1), L)]
        out_v[pl.ds(row0, L)] = acc

    fire(0, 0)

    def pair_body(h, carry):
        g0 = 2 * h
        fire(g0 + 1, 1)
        drain(0)
        compute(g0, 0)
        fire(lax.rem(g0 + 2, NG), 0)
        drain(1)
        compute(g0 + 1, 1)
        return carry

    lax.fori_loop(0, NG // 2, pair_body, 0)
    drain(0)

    pltpu.sync_copy(out_v, out_hbm.at[pl.ds(base, BPW)])


def kernel(users, items, user_table, item_table):
    ut3 = user_table.reshape(NT, 8, D)
    it3 = item_table.reshape(NT, 8, D)
    return _sc_forward(users, items, ut3, it3)
